# Initial kernel scaffold; baseline (speedup 1.0000x reference)
import jax
import jax.numpy as jnp
from jax.experimental import pallas as pl


def kernel(x, keys, values, in_proj_weight, in_proj_bias, out_proj_weight, out_proj_bias, ln_weight, ln_bias):
    raise NotImplementedError("write your pallas kernel here")


# speedup vs baseline: 14.6300x; 14.6300x over previous
import jax
import jax.numpy as jnp
from jax.experimental import pallas as pl

POOL = 64
PLEN = 8
TOPK = 8
HEADS = 4
BLOCK_B = 256


def _prep_kernel(keys_ref, vals_ref, wk_t_ref, bk_ref, wv_t_ref, bv_ref,
                 kn_ref, vk_ref, vv_ref):
    k = keys_ref[...]
    n = jnp.sqrt(jnp.sum(k * k, axis=1, keepdims=True))
    kn_ref[...] = k / jnp.maximum(n, 1e-12)
    v = vals_ref[...]
    vk_ref[...] = jnp.dot(v, wk_t_ref[...], preferred_element_type=jnp.float32) + bk_ref[...]
    vv_ref[...] = jnp.dot(v, wv_t_ref[...], preferred_element_type=jnp.float32) + bv_ref[...]


def _main_kernel(x_ref, kn_ref, vk_ref, vv_ref, wq_t_ref, bq_ref,
                 wo_t_ref, bo_ref, lnw_ref, lnb_ref, out_ref):
    x = x_ref[...]
    D = x.shape[1]
    dh = D // HEADS
    n = jnp.sqrt(jnp.sum(x * x, axis=1, keepdims=True))
    xn = x / jnp.maximum(n, 1e-12)
    sim = jax.lax.dot_general(xn, kn_ref[...], (((1,), (1,)), ((), ())),
                              preferred_element_type=jnp.float32)

    a = sim[:, None, :]
    b = sim[:, :, None]
    ii = jax.lax.broadcasted_iota(jnp.int32, (1, POOL, POOL), 2)
    jj = jax.lax.broadcasted_iota(jnp.int32, (1, POOL, POOL), 1)
    beats = jnp.where((a > b) | ((a == b) & (ii < jj)), 1.0, 0.0)
    cnt = jnp.sum(beats, axis=2)
    cnt_full = jnp.concatenate([cnt] * PLEN, axis=1)
    mask_full = cnt_full < float(TOPK)

    qh = jnp.dot(x, wq_t_ref[...], preferred_element_type=jnp.float32) + bq_ref[...]
    vk = vk_ref[...]
    vv = vv_ref[...]
    scale = 1.0 / jnp.sqrt(jnp.float32(dh))
    ctxs = []
    for h in range(HEADS):
        sl = slice(h * dh, (h + 1) * dh)
        lg = jax.lax.dot_general(qh[:, sl], vk[:, sl], (((1,), (1,)), ((), ())),
                                 preferred_element_type=jnp.float32) * scale
        lg = jnp.where(mask_full, lg, -1e30)
        m = jnp.max(lg, axis=1, keepdims=True)
        e = jnp.exp(lg - m)
        s = jnp.sum(e, axis=1, keepdims=True)
        ctxs.append(jnp.dot(e / s, vv[:, sl], preferred_element_type=jnp.float32))
    ctx = jnp.concatenate(ctxs, axis=1)

    y = x + jnp.dot(ctx, wo_t_ref[...], preferred_element_type=jnp.float32) + bo_ref[...]
    mu = jnp.mean(y, axis=1, keepdims=True)
    var = jnp.mean((y - mu) ** 2, axis=1, keepdims=True)
    out_ref[...] = (y - mu) / jnp.sqrt(var + 1e-5) * lnw_ref[...] + lnb_ref[...]


def kernel(x, keys, values, in_proj_weight, in_proj_bias, out_proj_weight,
           out_proj_bias, ln_weight, ln_bias):
    Bc, D = x.shape
    R = POOL * PLEN
    wq_t = in_proj_weight[:D].T
    wk_t = in_proj_weight[D:2 * D].T
    wv_t = in_proj_weight[2 * D:].T
    bq = in_proj_bias[:D].reshape(1, D)
    bk = in_proj_bias[D:2 * D].reshape(1, D)
    bv = in_proj_bias[2 * D:].reshape(1, D)
    wo_t = out_proj_weight.T
    bo = out_proj_bias.reshape(1, D)
    lnw = ln_weight.reshape(1, D)
    lnb = ln_bias.reshape(1, D)
    vals2d = values.transpose(1, 0, 2).reshape(R, D)

    kn, vk, vv = pl.pallas_call(
        _prep_kernel,
        out_shape=(
            jax.ShapeDtypeStruct((POOL, D), jnp.float32),
            jax.ShapeDtypeStruct((R, D), jnp.float32),
            jax.ShapeDtypeStruct((R, D), jnp.float32),
        ),
    )(keys, vals2d, wk_t, bk, wv_t, bv)

    nb = Bc // BLOCK_B
    full = lambda i: (0, 0)
    out = pl.pallas_call(
        _main_kernel,
        grid=(nb,),
        in_specs=[
            pl.BlockSpec((BLOCK_B, D), lambda i: (i, 0)),
            pl.BlockSpec((POOL, D), full),
            pl.BlockSpec((R, D), full),
            pl.BlockSpec((R, D), full),
            pl.BlockSpec((D, D), full),
            pl.BlockSpec((1, D), full),
            pl.BlockSpec((D, D), full),
            pl.BlockSpec((1, D), full),
            pl.BlockSpec((1, D), full),
            pl.BlockSpec((1, D), full),
        ],
        out_specs=pl.BlockSpec((BLOCK_B, D), lambda i: (i, 0)),
        out_shape=jax.ShapeDtypeStruct((Bc, D), jnp.float32),
    )(x, kn, vk, vv, wq_t, bq, wo_t, bo, lnw, lnb)
    return out


# speedup vs baseline: 15.6896x; 1.0724x over previous
import jax
import jax.numpy as jnp
from jax.experimental import pallas as pl

POOL = 64
PLEN = 8
TOPK = 8
HEADS = 4
BLOCK_B = 256


def _prep_kernel(keys_ref, vals_ref, wk_t_ref, bk_ref, wv_t_ref, bv_ref,
                 kn_ref, vk_ref, vv_ref):
    k = keys_ref[...]
    n = jnp.sqrt(jnp.sum(k * k, axis=1, keepdims=True))
    kn_ref[...] = k / jnp.maximum(n, 1e-12)
    v = vals_ref[...]
    vk = jnp.dot(v, wk_t_ref[...], preferred_element_type=jnp.float32) + bk_ref[...]
    vv = jnp.dot(v, wv_t_ref[...], preferred_element_type=jnp.float32) + bv_ref[...]
    vk_ref[...] = vk.astype(jnp.bfloat16)
    vv_ref[...] = vv.astype(jnp.bfloat16)


def _main_kernel(x_ref, kn_ref, vk_ref, vv_ref, wq_t_ref, bq_ref,
                 wo_t_ref, bo_ref, lnw_ref, lnb_ref, out_ref):
    x = x_ref[...]
    D = x.shape[1]
    dh = D // HEADS
    n = jnp.sqrt(jnp.sum(x * x, axis=1, keepdims=True))
    xn = x / jnp.maximum(n, 1e-12)
    sim = jax.lax.dot_general(xn, kn_ref[...], (((1,), (1,)), ((), ())),
                              preferred_element_type=jnp.float32)

    a = sim[:, None, :]
    b = sim[:, :, None]
    ii = jax.lax.broadcasted_iota(jnp.int32, (1, POOL, POOL), 2)
    jj = jax.lax.broadcasted_iota(jnp.int32, (1, POOL, POOL), 1)
    beats = jnp.where((a > b) | ((a == b) & (ii < jj)), 1.0, 0.0)
    cnt = jnp.sum(beats, axis=2)
    cnt_full = jnp.concatenate([cnt] * PLEN, axis=1)
    mask_full = cnt_full < float(TOPK)

    qh = jnp.dot(x.astype(jnp.bfloat16), wq_t_ref[...],
                 preferred_element_type=jnp.float32) + bq_ref[...]
    qh = qh.astype(jnp.bfloat16)
    vk = vk_ref[...]
    vv = vv_ref[...]
    scale = 1.0 / jnp.sqrt(jnp.float32(dh))
    ctxs = []
    for h in range(HEADS):
        sl = slice(h * dh, (h + 1) * dh)
        lg = jax.lax.dot_general(qh[:, sl], vk[:, sl], (((1,), (1,)), ((), ())),
                                 preferred_element_type=jnp.float32) * scale
        lg = jnp.where(mask_full, lg, -1e30)
        m = jnp.max(lg, axis=1, keepdims=True)
        e = jnp.exp(lg - m)
        s = jnp.sum(e, axis=1, keepdims=True)
        att = (e / s).astype(jnp.bfloat16)
        ctxs.append(jnp.dot(att, vv[:, sl], preferred_element_type=jnp.float32))
    ctx = jnp.concatenate(ctxs, axis=1).astype(jnp.bfloat16)

    attended = jnp.dot(ctx, wo_t_ref[...],
                       preferred_element_type=jnp.float32) + bo_ref[...]
    y = x + attended
    mu = jnp.mean(y, axis=1, keepdims=True)
    var = jnp.mean((y - mu) ** 2, axis=1, keepdims=True)
    out_ref[...] = (y - mu) / jnp.sqrt(var + 1e-5) * lnw_ref[...] + lnb_ref[...]


def kernel(x, keys, values, in_proj_weight, in_proj_bias, out_proj_weight,
           out_proj_bias, ln_weight, ln_bias):
    Bc, D = x.shape
    R = POOL * PLEN
    wq_t = in_proj_weight[:D].T.astype(jnp.bfloat16)
    wk_t = in_proj_weight[D:2 * D].T.astype(jnp.bfloat16)
    wv_t = in_proj_weight[2 * D:].T.astype(jnp.bfloat16)
    bq = in_proj_bias[:D].reshape(1, D)
    bk = in_proj_bias[D:2 * D].reshape(1, D)
    bv = in_proj_bias[2 * D:].reshape(1, D)
    wo_t = out_proj_weight.T.astype(jnp.bfloat16)
    bo = out_proj_bias.reshape(1, D)
    lnw = ln_weight.reshape(1, D)
    lnb = ln_bias.reshape(1, D)
    vals2d = values.transpose(1, 0, 2).reshape(R, D).astype(jnp.bfloat16)

    kn, vk, vv = pl.pallas_call(
        _prep_kernel,
        out_shape=(
            jax.ShapeDtypeStruct((POOL, D), jnp.float32),
            jax.ShapeDtypeStruct((R, D), jnp.bfloat16),
            jax.ShapeDtypeStruct((R, D), jnp.bfloat16),
        ),
    )(keys, vals2d, wk_t, bk, wv_t, bv)

    nb = Bc // BLOCK_B
    full = lambda i: (0, 0)
    out = pl.pallas_call(
        _main_kernel,
        grid=(nb,),
        in_specs=[
            pl.BlockSpec((BLOCK_B, D), lambda i: (i, 0)),
            pl.BlockSpec((POOL, D), full),
            pl.BlockSpec((R, D), full),
            pl.BlockSpec((R, D), full),
            pl.BlockSpec((D, D), full),
            pl.BlockSpec((1, D), full),
            pl.BlockSpec((D, D), full),
            pl.BlockSpec((1, D), full),
            pl.BlockSpec((1, D), full),
            pl.BlockSpec((1, D), full),
        ],
        out_specs=pl.BlockSpec((BLOCK_B, D), lambda i: (i, 0)),
        out_shape=jax.ShapeDtypeStruct((Bc, D), jnp.float32),
    )(x, kn, vk, vv, wq_t, bq, wo_t, bo, lnw, lnb)
    return out


# speedup vs baseline: 29.4213x; 1.8752x over previous
import jax
import jax.numpy as jnp
from jax.experimental import pallas as pl

POOL = 64
PLEN = 8
TOPK = 8
HEADS = 4
BLOCK_B = 256


def _prep_kernel(keys_ref, vals_ref, wk_t_ref, bk_ref, wv_t_ref, bv_ref,
                 kn_ref, vk_ref, vv_ref):
    k = keys_ref[...]
    n = jnp.sqrt(jnp.sum(k * k, axis=1, keepdims=True))
    kn_ref[...] = k / jnp.maximum(n, 1e-12)
    v = vals_ref[...]
    vk = jnp.dot(v, wk_t_ref[...], preferred_element_type=jnp.float32) + bk_ref[...]
    vv = jnp.dot(v, wv_t_ref[...], preferred_element_type=jnp.float32) + bv_ref[...]
    vk_ref[...] = vk.astype(jnp.bfloat16)
    vv_ref[...] = vv.astype(jnp.bfloat16)


def _main_kernel(x_ref, kn_ref, vk_ref, vv_ref, wq_t_ref, bq_ref,
                 wo_t_ref, bo_ref, lnw_ref, lnb_ref, out_ref):
    x = x_ref[...]
    D = x.shape[1]
    dh = D // HEADS
    n = jnp.sqrt(jnp.sum(x * x, axis=1, keepdims=True))
    xn = x / jnp.maximum(n, 1e-12)
    simT = jax.lax.dot_general(kn_ref[...], xn, (((1,), (1,)), ((), ())),
                               preferred_element_type=jnp.float32)

    iota_p = jax.lax.broadcasted_iota(jnp.int32, simT.shape, 0)
    cur = simT
    sel = jnp.zeros_like(simT)
    for _ in range(TOPK):
        m = jnp.max(cur, axis=0, keepdims=True)
        idx = jnp.where(cur == m, iota_p, POOL)
        jmin = jnp.min(idx, axis=0, keepdims=True)
        hit = iota_p == jmin
        sel = jnp.where(hit, 1.0, sel)
        cur = jnp.where(hit, -jnp.inf, cur)
    mask_pool = sel.T
    mask_full = jnp.concatenate([mask_pool] * PLEN, axis=1) > 0.5

    qh = jnp.dot(x.astype(jnp.bfloat16), wq_t_ref[...],
                 preferred_element_type=jnp.float32) + bq_ref[...]
    qh = qh.astype(jnp.bfloat16)
    vk = vk_ref[...]
    vv = vv_ref[...]
    scale = 1.0 / jnp.sqrt(jnp.float32(dh))
    ctxs = []
    for h in range(HEADS):
        sl = slice(h * dh, (h + 1) * dh)
        lg = jax.lax.dot_general(qh[:, sl], vk[:, sl], (((1,), (1,)), ((), ())),
                                 preferred_element_type=jnp.float32) * scale
        lg = jnp.where(mask_full, lg, -1e30)
        m = jnp.max(lg, axis=1, keepdims=True)
        e = jnp.exp(lg - m)
        s = jnp.sum(e, axis=1, keepdims=True)
        att = (e / s).astype(jnp.bfloat16)
        ctxs.append(jnp.dot(att, vv[:, sl], preferred_element_type=jnp.float32))
    ctx = jnp.concatenate(ctxs, axis=1).astype(jnp.bfloat16)

    attended = jnp.dot(ctx, wo_t_ref[...],
                       preferred_element_type=jnp.float32) + bo_ref[...]
    y = x + attended
    mu = jnp.mean(y, axis=1, keepdims=True)
    var = jnp.mean((y - mu) ** 2, axis=1, keepdims=True)
    out_ref[...] = (y - mu) / jnp.sqrt(var + 1e-5) * lnw_ref[...] + lnb_ref[...]


def kernel(x, keys, values, in_proj_weight, in_proj_bias, out_proj_weight,
           out_proj_bias, ln_weight, ln_bias):
    Bc, D = x.shape
    R = POOL * PLEN
    wq_t = in_proj_weight[:D].T.astype(jnp.bfloat16)
    wk_t = in_proj_weight[D:2 * D].T.astype(jnp.bfloat16)
    wv_t = in_proj_weight[2 * D:].T.astype(jnp.bfloat16)
    bq = in_proj_bias[:D].reshape(1, D)
    bk = in_proj_bias[D:2 * D].reshape(1, D)
    bv = in_proj_bias[2 * D:].reshape(1, D)
    wo_t = out_proj_weight.T.astype(jnp.bfloat16)
    bo = out_proj_bias.reshape(1, D)
    lnw = ln_weight.reshape(1, D)
    lnb = ln_bias.reshape(1, D)
    vals2d = values.transpose(1, 0, 2).reshape(R, D).astype(jnp.bfloat16)

    kn, vk, vv = pl.pallas_call(
        _prep_kernel,
        out_shape=(
            jax.ShapeDtypeStruct((POOL, D), jnp.float32),
            jax.ShapeDtypeStruct((R, D), jnp.bfloat16),
            jax.ShapeDtypeStruct((R, D), jnp.bfloat16),
        ),
    )(keys, vals2d, wk_t, bk, wv_t, bv)

    nb = Bc // BLOCK_B
    full = lambda i: (0, 0)
    out = pl.pallas_call(
        _main_kernel,
        grid=(nb,),
        in_specs=[
            pl.BlockSpec((BLOCK_B, D), lambda i: (i, 0)),
            pl.BlockSpec((POOL, D), full),
            pl.BlockSpec((R, D), full),
            pl.BlockSpec((R, D), full),
            pl.BlockSpec((D, D), full),
            pl.BlockSpec((1, D), full),
            pl.BlockSpec((D, D), full),
            pl.BlockSpec((1, D), full),
            pl.BlockSpec((1, D), full),
            pl.BlockSpec((1, D), full),
        ],
        out_specs=pl.BlockSpec((BLOCK_B, D), lambda i: (i, 0)),
        out_shape=jax.ShapeDtypeStruct((Bc, D), jnp.float32),
    )(x, kn, vk, vv, wq_t, bq, wo_t, bo, lnw, lnb)
    return out
